# baseline (speedup 1.0000x reference)
import functools

import jax
import jax.numpy as jnp
from jax.experimental import pallas as pl

IMAGE_SIZE = 512
PATCH = 16
CODEBOOK_SIZE = 8192
CODE_DIM = 32
IN_CH = 3

GRID_H = IMAGE_SIZE // PATCH
GRID_W = IMAGE_SIZE // PATCH
PH_TILE = 32
N_TILE = PH_TILE * GRID_W
FEAT = IN_CH * PATCH * PATCH


def _vq_kernel(x_ref, w_ref, cb_ref, c2_ref, out_ref):
    v = x_ref[0].reshape(IN_CH, PH_TILE, PATCH, IMAGE_SIZE)
    pieces = []
    for c in range(IN_CH):
        for i in range(PATCH):
            slab = v[c, :, i, :].astype(jnp.bfloat16)
            st = jnp.transpose(slab)
            pieces.append(st.reshape(GRID_W, PATCH, PH_TILE))
    stack = jnp.concatenate(pieces, axis=1)
    pb = jnp.transpose(stack, (0, 2, 1)).reshape(N_TILE, FEAT)

    z = jnp.dot(pb, w_ref[...], preferred_element_type=jnp.float32)
    z2 = jnp.sum(z * z, axis=1, keepdims=True)
    zb2 = (2.0 * z).astype(jnp.bfloat16)

    dn = (((1,), (1,)), ((), ()))
    KC = CODEBOOK_SIZE // 4
    run_min = None
    run_arg = None
    for kc in range(4):
        cbc = cb_ref[pl.ds(kc * KC, KC), :]
        s2 = jax.lax.dot_general(zb2, cbc, dn,
                                 preferred_element_type=jnp.float32)
        c2c = c2_ref[:, pl.ds(kc * KC, KC)]
        d = (z2 - s2) + c2c
        cmin = jnp.min(d, axis=1, keepdims=True)
        carg = jnp.argmin(d, axis=1)[:, None] + kc * KC
        if run_min is None:
            run_min, run_arg = cmin, carg
        else:
            better = cmin < run_min
            run_min = jnp.where(better, cmin, run_min)
            run_arg = jnp.where(better, carg, run_arg)
    arg = run_arg[:, 0]
    out_ref[...] = jnp.transpose(arg.reshape(GRID_W, PH_TILE))


@jax.jit
def _encode(x, W, codebook):
    B = x.shape[0]
    n_total = B * GRID_H * GRID_W
    c2 = jnp.sum(codebook * codebook, axis=-1)[None, :]
    wb = W.astype(jnp.bfloat16)
    cbb = codebook.astype(jnp.bfloat16)
    steps_per_img = GRID_H // PH_TILE
    grid = (B * steps_per_img,)
    out = pl.pallas_call(
        _vq_kernel,
        grid=grid,
        in_specs=[
            pl.BlockSpec(
                (1, IN_CH, PH_TILE * PATCH, IMAGE_SIZE),
                lambda i: (i // steps_per_img, 0, i % steps_per_img, 0),
            ),
            pl.BlockSpec((FEAT, CODE_DIM), lambda i: (0, 0)),
            pl.BlockSpec((CODEBOOK_SIZE, CODE_DIM), lambda i: (0, 0)),
            pl.BlockSpec((1, CODEBOOK_SIZE), lambda i: (0, 0)),
        ],
        out_specs=pl.BlockSpec((PH_TILE, GRID_W), lambda i: (i, 0)),
        out_shape=jax.ShapeDtypeStruct((B * GRID_H, GRID_W), jnp.int32),
    )(x, wb, cbb, c2)
    return out.reshape(B, GRID_H * GRID_W)


def kernel(x, W, codebook):
    indices = _encode(x, W, codebook)
    return (indices, GRID_H, GRID_W)


# speedup vs baseline: 1.2834x; 1.2834x over previous
import functools

import jax
import jax.numpy as jnp
from jax.experimental import pallas as pl

IMAGE_SIZE = 512
PATCH = 16
CODEBOOK_SIZE = 8192
CODE_DIM = 32
IN_CH = 3

GRID_H = IMAGE_SIZE // PATCH
GRID_W = IMAGE_SIZE // PATCH
PH_TILE = 32
N_TILE = PH_TILE * GRID_W
FEAT = IN_CH * PATCH * PATCH


def _vq_kernel(x_ref, w_ref, cb_ref, c2_ref, out_ref):
    v = x_ref[0].reshape(IN_CH, PH_TILE, PATCH, IMAGE_SIZE)
    pieces = []
    for c in range(IN_CH):
        for i in range(PATCH):
            slab = v[c, :, i, :].astype(jnp.bfloat16)
            st = jnp.transpose(slab)
            pieces.append(st.reshape(GRID_W, PATCH, PH_TILE))
    stack = jnp.concatenate(pieces, axis=1)
    pb = jnp.transpose(stack, (0, 2, 1)).reshape(N_TILE, FEAT)

    z = jnp.dot(pb, w_ref[...], preferred_element_type=jnp.float32)
    z2 = jnp.sum(z * z, axis=1, keepdims=True)
    zb2 = (2.0 * z).astype(jnp.bfloat16)

    dn = (((1,), (1,)), ((), ()))
    KC = CODEBOOK_SIZE // 2
    halves = []
    for kc in range(2):
        cbc = cb_ref[pl.ds(kc * KC, KC), :]
        s2 = jax.lax.dot_general(zb2, cbc, dn,
                                 preferred_element_type=jnp.float32)
        c2c = c2_ref[:, pl.ds(kc * KC, KC)]
        halves.append((z2 - s2) + c2c)
    d = jnp.concatenate(halves, axis=1)
    arg = jnp.argmin(d, axis=1)
    out_ref[...] = jnp.transpose(arg.reshape(GRID_W, PH_TILE))


@jax.jit
def _encode(x, W, codebook):
    B = x.shape[0]
    n_total = B * GRID_H * GRID_W
    c2 = jnp.sum(codebook * codebook, axis=-1)[None, :]
    wb = W.astype(jnp.bfloat16)
    cbb = codebook.astype(jnp.bfloat16)
    steps_per_img = GRID_H // PH_TILE
    grid = (B * steps_per_img,)
    out = pl.pallas_call(
        _vq_kernel,
        grid=grid,
        in_specs=[
            pl.BlockSpec(
                (1, IN_CH, PH_TILE * PATCH, IMAGE_SIZE),
                lambda i: (i // steps_per_img, 0, i % steps_per_img, 0),
            ),
            pl.BlockSpec((FEAT, CODE_DIM), lambda i: (0, 0)),
            pl.BlockSpec((CODEBOOK_SIZE, CODE_DIM), lambda i: (0, 0)),
            pl.BlockSpec((1, CODEBOOK_SIZE), lambda i: (0, 0)),
        ],
        out_specs=pl.BlockSpec((PH_TILE, GRID_W), lambda i: (i, 0)),
        out_shape=jax.ShapeDtypeStruct((B * GRID_H, GRID_W), jnp.int32),
    )(x, wb, cbb, c2)
    return out.reshape(B, GRID_H * GRID_W)


def kernel(x, W, codebook):
    indices = _encode(x, W, codebook)
    return (indices, GRID_H, GRID_W)
